# baseline (speedup 1.0000x reference)
import functools

import jax
import jax.numpy as jnp
from jax import lax
from jax.experimental import pallas as pl
from jax.experimental.pallas import tpu as pltpu
from jax.experimental.pallas import tpu_sc as plsc

_N = 10000
_IN = 128
_HID = 256
_OUT = 4
_NG = 128
_E2 = 640000
_NPAD = 10240
_NC, _NS = 2, 16
_NW = _NC * _NS
_EPT = _E2 // _NW
_G = 80
_NCH = _EPT // _G
_RPT = _NPAD // _NS
_TRASH = _NPAD - 8


def _tc_prep_body(x_ref, deg_ref, y_ref):
    dinv = lax.rsqrt(deg_ref[...])
    y_ref[...] = x_ref[...] * dinv


def _tc_prep(x_pad, degb):
    return pl.pallas_call(
        _tc_prep_body,
        grid=(_NPAD // 256,),
        in_specs=[
            pl.BlockSpec((256, _IN), lambda i: (i, 0)),
            pl.BlockSpec((256, _IN), lambda i: (i, 0)),
        ],
        out_specs=pl.BlockSpec((256, _IN), lambda i: (i, 0)),
        out_shape=jax.ShapeDtypeStruct((_NPAD, _IN), jnp.float32),
    )(x_pad, degb)


def _sc_agg_body(y_hbm, src_hbm, dst_hbm, ext_hbm, out_hbm,
                 src_v, dst_v, ybuf0, ybuf1, accbuf, fbuf, fidx, prevb,
                 exts, extd, extb, extv, sem0, sem1):
    c = lax.axis_index("c")
    s = lax.axis_index("s")
    wid = c * _NS + s
    e0 = pl.multiple_of(wid * _EPT, 16)
    it = lax.iota(jnp.int32, 16)
    zv = jnp.zeros((16,), jnp.float32)

    pltpu.sync_copy(src_hbm.at[pl.ds(e0, _EPT)], src_v)
    pltpu.sync_copy(dst_hbm.at[pl.ds(e0, _EPT)], dst_v)
    pltpu.sync_copy(src_hbm.at[pl.ds(pl.multiple_of(jnp.maximum(e0 - 16, 0), 16), 16)], prevb)
    for i in range(8):
        accbuf[0, pl.ds(16 * i, 16)] = zv
    fidx[0, pl.ds(0, 16)] = jnp.full((16,), _TRASH, jnp.int32)

    def _push(cs, sl):
        for i in range(8):
            fbuf[sl, pl.ds(16 * i, 16)] = accbuf[0, pl.ds(16 * i, 16)]
        fv = fidx[0, pl.ds(0, 16)]
        fidx[0, pl.ds(0, 16)] = jnp.where(it == sl, cs, fv)
        for i in range(8):
            accbuf[0, pl.ds(16 * i, 16)] = zv

    def _zero_acc():
        for i in range(8):
            accbuf[0, pl.ds(16 * i, 16)] = zv

    def _batch_dma():
        pltpu.sync_copy(fbuf, out_hbm.at[fidx.at[0]])
        fidx[0, pl.ds(0, 16)] = jnp.full((16,), _TRASH, jnp.int32)

    def _desc(k, buf, sem):
        return pltpu.make_async_copy(
            y_hbm.at[dst_v.at[pl.ds(pl.multiple_of(k * _G, 16), _G)]],
            buf, sem)

    def _process(ybuf, k, carry):
        def _group(g, cr):
            cur_s, slot, owned = cr
            gbase = pl.multiple_of(k * _G + g * 16, 16)
            sv = src_v[pl.ds(gbase, 16)]
            anyb = cur_s != sv[15]

            def _slow(op):
                cs, sl, ow = op
                for e in range(16):
                    s_e = sv[e]
                    new = s_e != cs

                    @pl.when(new & (ow != 0))
                    def _():
                        _push(cs, sl)

                    @pl.when(new & (ow == 0))
                    def _():
                        _zero_acc()

                    sl2 = jnp.where(new & (ow != 0), sl + 1, sl)

                    @pl.when(sl2 == 16)
                    def _():
                        _batch_dma()

                    sl = jnp.where(sl2 == 16, jnp.int32(0), sl2)
                    ow = jnp.where(new, jnp.int32(1), ow)
                    cs = s_e
                    for i in range(8):
                        plsc.addupdate(accbuf.at[0, pl.ds(16 * i, 16)],
                                       ybuf[g * 16 + e, pl.ds(16 * i, 16)])
                return (cs, sl, ow)

            def _fast(op):
                for e in range(16):
                    for i in range(8):
                        plsc.addupdate(accbuf.at[0, pl.ds(16 * i, 16)],
                                       ybuf[g * 16 + e, pl.ds(16 * i, 16)])
                return op

            return lax.cond(anyb, _slow, _fast, (cur_s, slot, owned))

        return lax.fori_loop(0, _G // 16, _group, carry)

    def _pair(j, carry):
        _desc(2 * j, ybuf0, sem0).wait()
        carry = _process(ybuf0, 2 * j, carry)

        @pl.when(j < _NCH // 2 - 1)
        def _():
            _desc(2 * j + 2, ybuf0, sem0).start()

        _desc(2 * j + 1, ybuf1, sem1).wait()
        carry = _process(ybuf1, 2 * j + 1, carry)

        @pl.when(j < _NCH // 2 - 1)
        def _():
            _desc(2 * j + 3, ybuf1, sem1).start()

        return carry

    v0 = src_v[pl.ds(0, 16)]
    pv = prevb[pl.ds(0, 16)]
    owned0 = jnp.where((wid == 0) | (v0[0] != pv[15]), jnp.int32(1),
                       jnp.int32(0))
    _desc(0, ybuf0, sem0).start()
    _desc(1, ybuf1, sem1).start()
    cur_s, slot, owned = lax.fori_loop(
        0, _NCH // 2, _pair, (v0[0], jnp.int32(0), owned0))

    pltpu.sync_copy(ext_hbm, extv)
    elv = extv[wid, pl.ds(0, 16)]
    el = elv[0]
    next_ch = jnp.where(owned != 0, (el + 15) // 16, jnp.int32(0))

    def _ext_body(j, carry):
        pos = pl.multiple_of(e0 + _EPT + 16 * j, 16)
        pltpu.sync_copy(dst_hbm.at[pl.ds(pos, 16)], extd)
        pltpu.async_copy(y_hbm.at[extd.at[pl.ds(0, 16)]], extb, sem0).wait()
        for e in range(16):
            valid = (16 * j + e) < el

            @pl.when(valid)
            def _():
                for i in range(8):
                    plsc.addupdate(accbuf.at[0, pl.ds(16 * i, 16)],
                                   extb[e, pl.ds(16 * i, 16)])
        return carry

    lax.fori_loop(0, next_ch, _ext_body, 0)

    @pl.when(owned != 0)
    def _():
        _push(cur_s, slot)

    pltpu.sync_copy(fbuf, out_hbm.at[fidx.at[0]])


def _sc_agg(y, src, dst, ext16):
    mesh = plsc.VectorSubcoreMesh(core_axis_name="c", subcore_axis_name="s")
    f = functools.partial(
        pl.kernel,
        out_type=jax.ShapeDtypeStruct((_NPAD, _IN), jnp.float32),
        mesh=mesh,
        scratch_types=[
            pltpu.VMEM((_EPT,), jnp.int32),
            pltpu.VMEM((_EPT,), jnp.int32),
            pltpu.VMEM((_G, _IN), jnp.float32),
            pltpu.VMEM((_G, _IN), jnp.float32),
            pltpu.VMEM((1, _IN), jnp.float32),
            pltpu.VMEM((16, _IN), jnp.float32),
            pltpu.VMEM((1, 16), jnp.int32),
            pltpu.VMEM((16,), jnp.int32),
            pltpu.VMEM((16,), jnp.int32),
            pltpu.VMEM((16,), jnp.int32),
            pltpu.VMEM((16, _IN), jnp.float32),
            pltpu.VMEM((_NW, 16), jnp.int32),
            pltpu.SemaphoreType.DMA,
            pltpu.SemaphoreType.DMA,
        ],
    )(_sc_agg_body)
    return f(y, src, dst, ext16)


def _tc_conv_body(p0_ref, x_ref, deg_ref, wc_ref, bc_ref, h_ref):
    degb = deg_ref[...]
    dinv = lax.rsqrt(degb)
    part = jnp.where(degb > 1.0, p0_ref[...], 0.0)
    pre = (part + dinv * x_ref[...]) * dinv
    acc = jnp.dot(pre, wc_ref[...], preferred_element_type=jnp.float32)
    h_ref[...] = jnp.maximum(acc + bc_ref[0:1, :], 0.0)


def _tc_conv(p0, x_pad, degb, Wc, bc2d):
    return pl.pallas_call(
        _tc_conv_body,
        grid=(_NPAD // 256,),
        in_specs=[
            pl.BlockSpec((256, _IN), lambda i: (i, 0)),
            pl.BlockSpec((256, _IN), lambda i: (i, 0)),
            pl.BlockSpec((256, _IN), lambda i: (i, 0)),
            pl.BlockSpec((_IN, _HID), lambda i: (0, 0)),
            pl.BlockSpec((8, _HID), lambda i: (0, 0)),
        ],
        out_specs=pl.BlockSpec((256, _HID), lambda i: (i, 0)),
        out_shape=jax.ShapeDtypeStruct((_NPAD, _HID), jnp.float32),
    )(p0, x_pad, degb, Wc, bc2d)


def _sc_pool_body(h_hbm, rpb_hbm, root_hbm, x_hbm, pooled_hbm, xroot_hbm,
                  rpb_v, hbuf, groupbuf, rootv, xrbuf, semp, semr):
    c = lax.axis_index("c")
    s = lax.axis_index("s")
    wid = c * _NS + s
    pltpu.sync_copy(rpb_hbm, rpb_v)
    rv = rpb_v[pl.ds(pl.multiple_of(8 * (wid // 2), 8), 16)]
    odd = (wid % 2) == 1
    ninf = jnp.full((16,), -jnp.inf, jnp.float32)

    for g in range(4):
        gs = jnp.where(odd, rv[g + 4], rv[g])
        ge = jnp.where(odd, rv[g + 5], rv[g + 1])
        base8 = pl.multiple_of((gs // 8) * 8, 8)
        nch = (ge - base8 + 63) // 64

        def _ch(jc, accs):
            b = pl.multiple_of(base8 + jc * 64, 8)
            pltpu.async_copy(h_hbm.at[pl.ds(b, 64)], hbuf, semp).wait()

            def _row(r, a2):
                rowi = b + r
                valid = (rowi >= gs) & (rowi < ge)
                outs = []
                for i in range(16):
                    v = hbuf[r, pl.ds(16 * i, 16)]
                    outs.append(jnp.where(valid, jnp.maximum(a2[i], v),
                                          a2[i]))
                return tuple(outs)

            return lax.fori_loop(0, 64, _row, accs)

        accs = lax.fori_loop(0, nch, _ch, (ninf,) * 16)
        for i in range(16):
            groupbuf[g, pl.ds(16 * i, 16)] = accs[i]

    zv = jnp.zeros((16,), jnp.float32)
    for g in range(4, 8):
        for i in range(16):
            groupbuf[g, pl.ds(16 * i, 16)] = zv
    pltpu.sync_copy(groupbuf, pooled_hbm.at[pl.ds(8 * wid, 8)])

    @pl.when(wid == 0)
    def _():
        pltpu.sync_copy(root_hbm, rootv)
        pltpu.async_copy(x_hbm.at[rootv], xrbuf, semr).wait()
        pltpu.sync_copy(xrbuf, xroot_hbm)


def _sc_pool(h, rpb_pad, root, x_pad):
    mesh = plsc.VectorSubcoreMesh(core_axis_name="c", subcore_axis_name="s")
    f = functools.partial(
        pl.kernel,
        out_type=(jax.ShapeDtypeStruct((8 * _NW, _HID), jnp.float32),
                  jax.ShapeDtypeStruct((_NG, _IN), jnp.float32)),
        mesh=mesh,
        scratch_types=[
            pltpu.VMEM((144,), jnp.int32),
            pltpu.VMEM((64, _HID), jnp.float32),
            pltpu.VMEM((8, _HID), jnp.float32),
            pltpu.VMEM((_NG,), jnp.int32),
            pltpu.VMEM((_NG, _IN), jnp.float32),
            pltpu.SemaphoreType.DMA,
            pltpu.SemaphoreType.DMA,
        ],
    )(_sc_pool_body)
    return f(h, rpb_pad, root, x_pad)


def _tc_head_body(xroot_ref, pooled_ref, w0_ref, b0_ref, w1_ref, b1_ref,
                  w2_ref, b2_ref, out_ref):
    news = jnp.maximum(
        jnp.dot(xroot_ref[...], w0_ref[...],
                preferred_element_type=jnp.float32) + b0_ref[0:1, :], 0.0)
    h2 = jnp.maximum(
        jnp.dot(news, w1_ref[0:256, :], preferred_element_type=jnp.float32)
        + jnp.dot(pooled_ref[...], w1_ref[256:512, :],
                  preferred_element_type=jnp.float32) + b1_ref[0:1, :], 0.0)
    z = jnp.dot(h2, w2_ref[...],
                preferred_element_type=jnp.float32) + b2_ref[0:1, :]
    ci = lax.broadcasted_iota(jnp.int32, (_NG, 128), 1)
    zm = jnp.where(ci < _OUT, z, -jnp.inf)
    mx = jnp.max(zm, axis=1, keepdims=True)
    ex = jnp.where(ci < _OUT, jnp.exp(z - mx), 0.0)
    lse = jnp.log(jnp.sum(ex, axis=1, keepdims=True))
    out_ref[...] = z - mx - lse


def _tc_head(xroot, pooled, W0, b02d, W1, b12d, W2p, b22d):
    return pl.pallas_call(
        _tc_head_body,
        out_shape=jax.ShapeDtypeStruct((_NG, 128), jnp.float32),
    )(xroot, pooled, W0, b02d, W1, b12d, W2p, b22d)


def kernel(x, edge_index, batch, Wc, bc, W0, b0, W1, b1, W2, b2):
    n = x.shape[0]
    row = jnp.concatenate([edge_index[0], edge_index[1]])
    col = jnp.concatenate([edge_index[1], edge_index[0]])
    keys = jnp.sort(row * n + col)
    dup = jnp.concatenate(
        [jnp.zeros((1,), bool), keys[1:] == keys[:-1]])
    src = (keys // n).astype(jnp.int32)
    dst = jnp.where(dup, n, keys % n).astype(jnp.int32)

    rp = jnp.searchsorted(src, jnp.arange(n + 1, dtype=jnp.int32)
                          ).astype(jnp.int32)
    cs = jnp.concatenate([jnp.zeros((1,), jnp.float32),
                          jnp.cumsum((~dup).astype(jnp.float32))])
    deg = cs[rp[1:]] - cs[rp[:-1]] + 1.0
    deg_pad = jnp.concatenate([deg, jnp.ones((_NPAD - n,), jnp.float32)])
    degb = jnp.broadcast_to(deg_pad[:, None], (_NPAD, _IN))
    x_pad = jnp.concatenate(
        [x, jnp.zeros((_NPAD - n, _IN), jnp.float32)], axis=0)

    e1s = (jnp.arange(_NW, dtype=jnp.int32) + 1) * _EPT
    s_last = src[e1s - 1]
    ext_len = jnp.maximum(rp[s_last + 1] - e1s, 0)
    ext16 = jnp.broadcast_to(ext_len[:, None], (_NW, 16))
    src_p = jnp.concatenate([src, jnp.zeros((16,), jnp.int32)])
    dst_p = jnp.concatenate([dst, jnp.full((16,), n, jnp.int32)])

    y = _tc_prep(x_pad, degb)
    part = _sc_agg(y, src_p, dst_p, ext16)
    bc2d = jnp.broadcast_to(bc, (8, _HID))
    h = _tc_conv(part, x_pad, degb, Wc, bc2d)

    rpb = jnp.searchsorted(batch, jnp.arange(_NG + 1, dtype=jnp.int32)
                           ).astype(jnp.int32)
    rpb_pad = jnp.concatenate([rpb, jnp.zeros((15,), jnp.int32)])
    diff = batch[1:] - batch[:-1]
    changes = jnp.nonzero(diff, size=_NG - 1)[0]
    root = jnp.concatenate(
        [jnp.zeros((1,), changes.dtype), changes + 1]).astype(jnp.int32)

    pooled_pad, xroot = _sc_pool(h, rpb_pad, root, x_pad)
    pooled = pooled_pad.reshape(_NW, 8, _HID)[:, :4].reshape(_NG, _HID)

    b02d = jnp.broadcast_to(b0, (8, _HID))
    b12d = jnp.broadcast_to(b1, (8, _HID))
    W2p = jnp.pad(W2, ((0, 0), (0, 128 - _OUT)))
    b22d = jnp.broadcast_to(jnp.pad(b2, (0, 128 - _OUT)), (8, 128))
    out = _tc_head(xroot, pooled, W0, b02d, W1, b12d, W2p, b22d)
    return out[:, :_OUT]


# speedup vs baseline: 1.1682x; 1.1682x over previous
import functools

import jax
import jax.numpy as jnp
from jax import lax
from jax.experimental import pallas as pl
from jax.experimental.pallas import tpu as pltpu
from jax.experimental.pallas import tpu_sc as plsc

_N = 10000
_IN = 128
_HID = 256
_OUT = 4
_NG = 128
_E2 = 640000
_NPAD = 10240
_NC, _NS = 2, 16
_NW = _NC * _NS
_EPT = _E2 // _NW
_G = 80
_NCH = _EPT // _G
_RPT = _NPAD // _NS
_TRASH = _NPAD - 8


def _tc_prep_body(x_ref, deg_ref, y_ref):
    dinv = lax.rsqrt(deg_ref[...])
    y_ref[...] = x_ref[...] * dinv


def _tc_prep(x_pad, degb):
    return pl.pallas_call(
        _tc_prep_body,
        grid=(_NPAD // 256,),
        in_specs=[
            pl.BlockSpec((256, _IN), lambda i: (i, 0)),
            pl.BlockSpec((256, _IN), lambda i: (i, 0)),
        ],
        out_specs=pl.BlockSpec((256, _IN), lambda i: (i, 0)),
        out_shape=jax.ShapeDtypeStruct((_NPAD, _IN), jnp.float32),
    )(x_pad, degb)


def _sc_agg_body(y_hbm, src_hbm, dst_hbm, ext_hbm, out_hbm,
                 src_v, dst_v, ybuf0, ybuf1, accbuf, fbuf, fidx, prevb,
                 exts, extd, extb, extv, sem0, sem1):
    c = lax.axis_index("c")
    s = lax.axis_index("s")
    wid = c * _NS + s
    e0 = pl.multiple_of(wid * _EPT, 16)
    it = lax.iota(jnp.int32, 16)
    zv = jnp.zeros((16,), jnp.float32)

    pltpu.sync_copy(src_hbm.at[pl.ds(e0, _EPT)], src_v)
    pltpu.sync_copy(dst_hbm.at[pl.ds(e0, _EPT)], dst_v)
    pltpu.sync_copy(src_hbm.at[pl.ds(pl.multiple_of(jnp.maximum(e0 - 16, 0), 16), 16)], prevb)
    for i in range(8):
        accbuf[0, pl.ds(16 * i, 16)] = zv
    fidx[0, pl.ds(0, 16)] = jnp.full((16,), _TRASH, jnp.int32)

    def _push(cs, sl):
        for i in range(8):
            fbuf[sl, pl.ds(16 * i, 16)] = accbuf[0, pl.ds(16 * i, 16)]
        fv = fidx[0, pl.ds(0, 16)]
        fidx[0, pl.ds(0, 16)] = jnp.where(it == sl, cs, fv)
        for i in range(8):
            accbuf[0, pl.ds(16 * i, 16)] = zv

    def _zero_acc():
        for i in range(8):
            accbuf[0, pl.ds(16 * i, 16)] = zv

    def _batch_dma():
        pltpu.sync_copy(fbuf, out_hbm.at[fidx.at[0]])
        fidx[0, pl.ds(0, 16)] = jnp.full((16,), _TRASH, jnp.int32)

    def _desc(k, buf, sem):
        return pltpu.make_async_copy(
            y_hbm.at[dst_v.at[pl.ds(pl.multiple_of(k * _G, 16), _G)]],
            buf, sem)

    def _process(ybuf, k, carry):
        def _group(g, cr):
            cur_s, slot, owned = cr
            gbase = pl.multiple_of(k * _G + g * 16, 16)
            sv = src_v[pl.ds(gbase, 16)]
            anyb = cur_s != sv[15]

            def _slow(op):
                cs, sl, ow = op
                run = [zv] * 8
                for e in range(16):
                    s_e = sv[e]
                    new = s_e != cs
                    run_c = run

                    @pl.when(new)
                    def _():
                        for i in range(8):
                            plsc.addupdate(accbuf.at[0, pl.ds(16 * i, 16)],
                                           run_c[i])

                    @pl.when(new & (ow != 0))
                    def _():
                        _push(cs, sl)

                    @pl.when(new & (ow == 0))
                    def _():
                        _zero_acc()

                    sl2 = jnp.where(new & (ow != 0), sl + 1, sl)

                    @pl.when(sl2 == 16)
                    def _():
                        _batch_dma()

                    sl = jnp.where(sl2 == 16, jnp.int32(0), sl2)
                    ow = jnp.where(new, jnp.int32(1), ow)
                    cs = s_e
                    rows = [ybuf[g * 16 + e, pl.ds(16 * i, 16)]
                            for i in range(8)]
                    run = [jnp.where(new, rows[i], run[i] + rows[i])
                           for i in range(8)]
                for i in range(8):
                    plsc.addupdate(accbuf.at[0, pl.ds(16 * i, 16)], run[i])
                return (cs, sl, ow)

            def _fast(op):
                for i in range(8):
                    vals = [ybuf[g * 16 + e, pl.ds(16 * i, 16)]
                            for e in range(16)]
                    while len(vals) > 1:
                        vals = [vals[j] + vals[j + 1]
                                for j in range(0, len(vals), 2)]
                    plsc.addupdate(accbuf.at[0, pl.ds(16 * i, 16)], vals[0])
                return op

            return lax.cond(anyb, _slow, _fast, (cur_s, slot, owned))

        return lax.fori_loop(0, _G // 16, _group, carry)

    def _pair(j, carry):
        _desc(2 * j, ybuf0, sem0).wait()
        carry = _process(ybuf0, 2 * j, carry)

        @pl.when(j < _NCH // 2 - 1)
        def _():
            _desc(2 * j + 2, ybuf0, sem0).start()

        _desc(2 * j + 1, ybuf1, sem1).wait()
        carry = _process(ybuf1, 2 * j + 1, carry)

        @pl.when(j < _NCH // 2 - 1)
        def _():
            _desc(2 * j + 3, ybuf1, sem1).start()

        return carry

    v0 = src_v[pl.ds(0, 16)]
    pv = prevb[pl.ds(0, 16)]
    owned0 = jnp.where((wid == 0) | (v0[0] != pv[15]), jnp.int32(1),
                       jnp.int32(0))
    _desc(0, ybuf0, sem0).start()
    _desc(1, ybuf1, sem1).start()
    cur_s, slot, owned = lax.fori_loop(
        0, _NCH // 2, _pair, (v0[0], jnp.int32(0), owned0))

    pltpu.sync_copy(ext_hbm, extv)
    elv = extv[wid, pl.ds(0, 16)]
    el = elv[0]
    next_ch = jnp.where(owned != 0, (el + 15) // 16, jnp.int32(0))

    def _ext_body(j, carry):
        pos = pl.multiple_of(e0 + _EPT + 16 * j, 16)
        pltpu.sync_copy(dst_hbm.at[pl.ds(pos, 16)], extd)
        pltpu.async_copy(y_hbm.at[extd.at[pl.ds(0, 16)]], extb, sem0).wait()
        for e in range(16):
            valid = (16 * j + e) < el

            @pl.when(valid)
            def _():
                for i in range(8):
                    plsc.addupdate(accbuf.at[0, pl.ds(16 * i, 16)],
                                   extb[e, pl.ds(16 * i, 16)])
        return carry

    lax.fori_loop(0, next_ch, _ext_body, 0)

    @pl.when(owned != 0)
    def _():
        _push(cur_s, slot)

    pltpu.sync_copy(fbuf, out_hbm.at[fidx.at[0]])


def _sc_agg(y, src, dst, ext16):
    mesh = plsc.VectorSubcoreMesh(core_axis_name="c", subcore_axis_name="s")
    f = functools.partial(
        pl.kernel,
        out_type=jax.ShapeDtypeStruct((_NPAD, _IN), jnp.float32),
        mesh=mesh,
        scratch_types=[
            pltpu.VMEM((_EPT,), jnp.int32),
            pltpu.VMEM((_EPT,), jnp.int32),
            pltpu.VMEM((_G, _IN), jnp.float32),
            pltpu.VMEM((_G, _IN), jnp.float32),
            pltpu.VMEM((1, _IN), jnp.float32),
            pltpu.VMEM((16, _IN), jnp.float32),
            pltpu.VMEM((1, 16), jnp.int32),
            pltpu.VMEM((16,), jnp.int32),
            pltpu.VMEM((16,), jnp.int32),
            pltpu.VMEM((16,), jnp.int32),
            pltpu.VMEM((16, _IN), jnp.float32),
            pltpu.VMEM((_NW, 16), jnp.int32),
            pltpu.SemaphoreType.DMA,
            pltpu.SemaphoreType.DMA,
        ],
    )(_sc_agg_body)
    return f(y, src, dst, ext16)


def _tc_conv_body(p0_ref, x_ref, deg_ref, wc_ref, bc_ref, h_ref):
    degb = deg_ref[...]
    dinv = lax.rsqrt(degb)
    part = jnp.where(degb > 1.0, p0_ref[...], 0.0)
    pre = (part + dinv * x_ref[...]) * dinv
    acc = jnp.dot(pre, wc_ref[...], preferred_element_type=jnp.float32)
    h_ref[...] = jnp.maximum(acc + bc_ref[0:1, :], 0.0)


def _tc_conv(p0, x_pad, degb, Wc, bc2d):
    return pl.pallas_call(
        _tc_conv_body,
        grid=(_NPAD // 256,),
        in_specs=[
            pl.BlockSpec((256, _IN), lambda i: (i, 0)),
            pl.BlockSpec((256, _IN), lambda i: (i, 0)),
            pl.BlockSpec((256, _IN), lambda i: (i, 0)),
            pl.BlockSpec((_IN, _HID), lambda i: (0, 0)),
            pl.BlockSpec((8, _HID), lambda i: (0, 0)),
        ],
        out_specs=pl.BlockSpec((256, _HID), lambda i: (i, 0)),
        out_shape=jax.ShapeDtypeStruct((_NPAD, _HID), jnp.float32),
    )(p0, x_pad, degb, Wc, bc2d)


def _sc_pool_body(h_hbm, rpb_hbm, root_hbm, x_hbm, pooled_hbm, xroot_hbm,
                  rpb_v, hbuf, groupbuf, rootv, xrbuf, semp, semr):
    c = lax.axis_index("c")
    s = lax.axis_index("s")
    wid = c * _NS + s
    pltpu.sync_copy(rpb_hbm, rpb_v)
    rv = rpb_v[pl.ds(pl.multiple_of(8 * (wid // 2), 8), 16)]
    odd = (wid % 2) == 1
    ninf = jnp.full((16,), -jnp.inf, jnp.float32)

    for g in range(4):
        gs = jnp.where(odd, rv[g + 4], rv[g])
        ge = jnp.where(odd, rv[g + 5], rv[g + 1])
        base8 = pl.multiple_of((gs // 8) * 8, 8)
        nch = (ge - base8 + 63) // 64

        def _ch(jc, accs):
            b = pl.multiple_of(base8 + jc * 64, 8)
            pltpu.async_copy(h_hbm.at[pl.ds(b, 64)], hbuf, semp).wait()

            def _row(r, a2):
                rowi = b + r
                valid = (rowi >= gs) & (rowi < ge)
                outs = []
                for i in range(16):
                    v = hbuf[r, pl.ds(16 * i, 16)]
                    outs.append(jnp.where(valid, jnp.maximum(a2[i], v),
                                          a2[i]))
                return tuple(outs)

            return lax.fori_loop(0, 64, _row, accs)

        accs = lax.fori_loop(0, nch, _ch, (ninf,) * 16)
        for i in range(16):
            groupbuf[g, pl.ds(16 * i, 16)] = accs[i]

    zv = jnp.zeros((16,), jnp.float32)
    for g in range(4, 8):
        for i in range(16):
            groupbuf[g, pl.ds(16 * i, 16)] = zv
    pltpu.sync_copy(groupbuf, pooled_hbm.at[pl.ds(8 * wid, 8)])

    @pl.when(wid == 0)
    def _():
        pltpu.sync_copy(root_hbm, rootv)
        pltpu.async_copy(x_hbm.at[rootv], xrbuf, semr).wait()
        pltpu.sync_copy(xrbuf, xroot_hbm)


def _sc_pool(h, rpb_pad, root, x_pad):
    mesh = plsc.VectorSubcoreMesh(core_axis_name="c", subcore_axis_name="s")
    f = functools.partial(
        pl.kernel,
        out_type=(jax.ShapeDtypeStruct((8 * _NW, _HID), jnp.float32),
                  jax.ShapeDtypeStruct((_NG, _IN), jnp.float32)),
        mesh=mesh,
        scratch_types=[
            pltpu.VMEM((144,), jnp.int32),
            pltpu.VMEM((64, _HID), jnp.float32),
            pltpu.VMEM((8, _HID), jnp.float32),
            pltpu.VMEM((_NG,), jnp.int32),
            pltpu.VMEM((_NG, _IN), jnp.float32),
            pltpu.SemaphoreType.DMA,
            pltpu.SemaphoreType.DMA,
        ],
    )(_sc_pool_body)
    return f(h, rpb_pad, root, x_pad)


def _tc_head_body(xroot_ref, pooled_ref, w0_ref, b0_ref, w1_ref, b1_ref,
                  w2_ref, b2_ref, out_ref):
    news = jnp.maximum(
        jnp.dot(xroot_ref[...], w0_ref[...],
                preferred_element_type=jnp.float32) + b0_ref[0:1, :], 0.0)
    h2 = jnp.maximum(
        jnp.dot(news, w1_ref[0:256, :], preferred_element_type=jnp.float32)
        + jnp.dot(pooled_ref[...], w1_ref[256:512, :],
                  preferred_element_type=jnp.float32) + b1_ref[0:1, :], 0.0)
    z = jnp.dot(h2, w2_ref[...],
                preferred_element_type=jnp.float32) + b2_ref[0:1, :]
    ci = lax.broadcasted_iota(jnp.int32, (_NG, 128), 1)
    zm = jnp.where(ci < _OUT, z, -jnp.inf)
    mx = jnp.max(zm, axis=1, keepdims=True)
    ex = jnp.where(ci < _OUT, jnp.exp(z - mx), 0.0)
    lse = jnp.log(jnp.sum(ex, axis=1, keepdims=True))
    out_ref[...] = z - mx - lse


def _tc_head(xroot, pooled, W0, b02d, W1, b12d, W2p, b22d):
    return pl.pallas_call(
        _tc_head_body,
        out_shape=jax.ShapeDtypeStruct((_NG, 128), jnp.float32),
    )(xroot, pooled, W0, b02d, W1, b12d, W2p, b22d)


def kernel(x, edge_index, batch, Wc, bc, W0, b0, W1, b1, W2, b2):
    n = x.shape[0]
    row = jnp.concatenate([edge_index[0], edge_index[1]])
    col = jnp.concatenate([edge_index[1], edge_index[0]])
    keys = jnp.sort(row * n + col)
    dup = jnp.concatenate(
        [jnp.zeros((1,), bool), keys[1:] == keys[:-1]])
    src = (keys // n).astype(jnp.int32)
    dst = jnp.where(dup, n, keys % n).astype(jnp.int32)

    rp = jnp.searchsorted(src, jnp.arange(n + 1, dtype=jnp.int32)
                          ).astype(jnp.int32)
    cs = jnp.concatenate([jnp.zeros((1,), jnp.float32),
                          jnp.cumsum((~dup).astype(jnp.float32))])
    deg = cs[rp[1:]] - cs[rp[:-1]] + 1.0
    deg_pad = jnp.concatenate([deg, jnp.ones((_NPAD - n,), jnp.float32)])
    degb = jnp.broadcast_to(deg_pad[:, None], (_NPAD, _IN))
    x_pad = jnp.concatenate(
        [x, jnp.zeros((_NPAD - n, _IN), jnp.float32)], axis=0)

    e1s = (jnp.arange(_NW, dtype=jnp.int32) + 1) * _EPT
    s_last = src[e1s - 1]
    ext_len = jnp.maximum(rp[s_last + 1] - e1s, 0)
    ext16 = jnp.broadcast_to(ext_len[:, None], (_NW, 16))
    src_p = jnp.concatenate([src, jnp.zeros((16,), jnp.int32)])
    dst_p = jnp.concatenate([dst, jnp.full((16,), n, jnp.int32)])

    y = _tc_prep(x_pad, degb)
    part = _sc_agg(y, src_p, dst_p, ext16)
    bc2d = jnp.broadcast_to(bc, (8, _HID))
    h = _tc_conv(part, x_pad, degb, Wc, bc2d)

    rpb = jnp.searchsorted(batch, jnp.arange(_NG + 1, dtype=jnp.int32)
                           ).astype(jnp.int32)
    rpb_pad = jnp.concatenate([rpb, jnp.zeros((15,), jnp.int32)])
    diff = batch[1:] - batch[:-1]
    changes = jnp.nonzero(diff, size=_NG - 1)[0]
    root = jnp.concatenate(
        [jnp.zeros((1,), changes.dtype), changes + 1]).astype(jnp.int32)

    pooled_pad, xroot = _sc_pool(h, rpb_pad, root, x_pad)
    pooled = pooled_pad.reshape(_NW, 8, _HID)[:, :4].reshape(_NG, _HID)

    b02d = jnp.broadcast_to(b0, (8, _HID))
    b12d = jnp.broadcast_to(b1, (8, _HID))
    W2p = jnp.pad(W2, ((0, 0), (0, 128 - _OUT)))
    b22d = jnp.broadcast_to(jnp.pad(b2, (0, 128 - _OUT)), (8, 128))
    out = _tc_head(xroot, pooled, W0, b02d, W1, b12d, W2p, b22d)
    return out[:, :_OUT]
